# Initial kernel scaffold; baseline (speedup 1.0000x reference)
import jax
import jax.numpy as jnp
from jax.experimental import pallas as pl


def kernel(x, weight):
    raise NotImplementedError("write your pallas kernel here")


# speedup vs baseline: 1.8894x; 1.8894x over previous
import functools

import jax
import jax.numpy as jnp
from jax import lax
from jax.experimental import pallas as pl
from jax.experimental.pallas import tpu as pltpu
from jax.experimental.pallas import tpu_sc as plsc

NC = 2
NS = 16
NW = NC * NS

CHUNK = 128
NBUF = 4


@functools.lru_cache(maxsize=None)
def _make_gather(B, V, D, b_per_w, n_chunks):
    mesh = plsc.VectorSubcoreMesh(core_axis_name="c", subcore_axis_name="s")
    nbuf = NBUF
    assert n_chunks % nbuf == 0 and n_chunks // nbuf >= 2
    n_blocks = n_chunks // nbuf
    row_bytes = CHUNK * D * 4

    @functools.partial(
        pl.kernel,
        out_type=jax.ShapeDtypeStruct((B, D), jnp.float32),
        mesh=mesh,
        compiler_params=pltpu.CompilerParams(use_tc_tiling_on_sc=False),
        scratch_types=[
            pltpu.VMEM((n_chunks, CHUNK), jnp.int32),
            [pltpu.VMEM((CHUNK, D), jnp.float32) for _ in range(nbuf)],
            pltpu.SemaphoreType.DMA,
        ],
    )
    def k(idx_hbm, table_hbm, out_hbm, idx_v, rows, gsem):
        wid = lax.axis_index("s") * NC + lax.axis_index("c")
        base = wid * b_per_w

        pltpu.sync_copy(idx_hbm.at[wid], idx_v)

        def start_gather(g, b):
            pltpu.async_copy(table_hbm.at[idx_v.at[g]], rows[b], gsem)

        def wait_gather(b):
            pltpu.make_async_copy(table_hbm.at[idx_v.at[0]], rows[b], gsem).wait()

        def copy_out(g, b):
            pltpu.sync_copy(rows[b], out_hbm.at[pl.ds(base + g * CHUNK, CHUNK)])

        for b in range(nbuf):
            start_gather(b, b)

        def body(blk, _):
            for b in range(nbuf):
                g = blk * nbuf + b
                wait_gather(b)
                copy_out(g, b)
                start_gather(g + nbuf, b)
            return _

        lax.fori_loop(0, n_blocks - 1, body, None)

        for b in range(nbuf):
            g = (n_blocks - 1) * nbuf + b
            wait_gather(b)
            copy_out(g, b)

    return k


def kernel(x, weight):
    BATCH, HIST = x.shape
    V, D = weight.shape
    B = BATCH * HIST
    b_per_w = B // NW
    n_chunks = b_per_w // CHUNK
    idx = x.reshape(NW, n_chunks, CHUNK).astype(jnp.int32)
    out = _make_gather(B, V, D, b_per_w, n_chunks)(idx, weight)
    return out.reshape(BATCH, HIST, D)
